# baseline (speedup 1.0000x reference)
import jax
import jax.numpy as jnp
import numpy as np
from jax.experimental import pallas as pl
from jax.experimental.pallas import tpu as pltpu

_N = 4096
_T = 1024
_NB = _N // _T
_PAIRS = [(i, j) for i in range(_NB) for j in range(_NB) if i <= j]
_P = len(_PAIRS)
_IJ = np.array([[p[0] for p in _PAIRS], [p[1] for p in _PAIRS]], dtype=np.int32)
_MARGIN = 1.0


def _tl_kernel(ij_ref, xi_ref, xj_ref, ti_ref, tj_ref, out_ref,
               pos_ref, neg_ref, pm_ref):
    p = pl.program_id(0)
    inf = jnp.float32(jnp.inf)
    iI = ij_ref[0, p]
    jJ = ij_ref[1, p]

    @pl.when(p == 0)
    def _init():
        pos_ref[...] = jnp.zeros_like(pos_ref)
        neg_ref[...] = jnp.full_like(neg_ref, inf)
        pm_ref[...] = jnp.full_like(pm_ref, inf)

    xi = xi_ref[...]
    xj = xj_ref[...]
    ti = ti_ref[...]
    tj = tj_ref[...]

    sq_i = jnp.sum(xi * xi, axis=1, keepdims=True)
    sq_j = jnp.sum(xj * xj, axis=1)[None, :]
    gram = jax.lax.dot_general(
        xi, xj, (((1,), (1,)), ((), ())),
        preferred_element_type=jnp.float32)
    d2 = (sq_i + sq_j) - (gram + gram)
    same = ti == tj

    posv = jnp.where(same, d2, 0.0)
    negv = jnp.where(same, inf, d2)
    pmv = jnp.where(same, d2, inf)

    def rowslice(ref, base):
        return ref[slice(0, 1), pl.ds(base, _T)]

    base_i = iI * _T
    pr = jnp.max(posv, axis=1, keepdims=True)
    nr = jnp.min(negv, axis=1, keepdims=True)

    @pl.when(iI == jJ)
    def _diag_row():
        pm0 = jnp.min(pmv, axis=1, keepdims=True)
        pms = jnp.min(jnp.where(pmv > pm0, pmv, inf),
                      axis=1, keepdims=True)
        st = jnp.transpose(jnp.concatenate([pr, nr, pms], axis=1))
        pos_ref[slice(0, 1), pl.ds(base_i, _T)] = jnp.maximum(
            rowslice(pos_ref, base_i), st[0:1, :])
        neg_ref[slice(0, 1), pl.ds(base_i, _T)] = jnp.minimum(
            rowslice(neg_ref, base_i), st[1:2, :])
        pm_ref[slice(0, 1), pl.ds(base_i, _T)] = jnp.minimum(
            rowslice(pm_ref, base_i), st[2:3, :])

    @pl.when(iI != jJ)
    def _offdiag():
        pmr = jnp.min(pmv, axis=1, keepdims=True)
        st = jnp.transpose(jnp.concatenate([pr, nr, pmr], axis=1))
        pos_ref[slice(0, 1), pl.ds(base_i, _T)] = jnp.maximum(
            rowslice(pos_ref, base_i), st[0:1, :])
        neg_ref[slice(0, 1), pl.ds(base_i, _T)] = jnp.minimum(
            rowslice(neg_ref, base_i), st[1:2, :])
        pm_ref[slice(0, 1), pl.ds(base_i, _T)] = jnp.minimum(
            rowslice(pm_ref, base_i), st[2:3, :])
        base_j = jJ * _T
        pc = jnp.max(posv, axis=0, keepdims=True)
        nc = jnp.min(negv, axis=0, keepdims=True)
        pmc = jnp.min(pmv, axis=0, keepdims=True)
        pos_ref[slice(0, 1), pl.ds(base_j, _T)] = jnp.maximum(
            rowslice(pos_ref, base_j), pc)
        neg_ref[slice(0, 1), pl.ds(base_j, _T)] = jnp.minimum(
            rowslice(neg_ref, base_j), nc)
        pm_ref[slice(0, 1), pl.ds(base_j, _T)] = jnp.minimum(
            rowslice(pm_ref, base_j), pmc)

    @pl.when(p == _P - 1)
    def _finish():
        dist_p = jnp.sqrt(jnp.maximum(pos_ref[...], 0.0))
        dist_n = jnp.sqrt(jnp.maximum(neg_ref[...], 0.0))
        top1_same = (pm_ref[...] < neg_ref[...]).astype(jnp.float32)
        diff = jnp.maximum(dist_p - dist_n + _MARGIN, 0.0)
        out_ref[...] = jnp.stack([
            jnp.sum(diff),
            jnp.sum(top1_same),
            jnp.sum((dist_n > dist_p).astype(jnp.float32)),
            jnp.sum((dist_n > dist_p + _MARGIN).astype(jnp.float32)),
            jnp.sum((diff != 0.0).astype(jnp.float32)),
            jnp.sum(dist_p),
            jnp.sum(dist_n),
            jnp.sum((dist_n - dist_p) / jnp.maximum(dist_p, dist_n)),
        ])[None, :]


@jax.jit
def _triplet_stats(x, t32):
    grid_spec = pltpu.PrefetchScalarGridSpec(
        num_scalar_prefetch=1,
        grid=(_P,),
        in_specs=[
            pl.BlockSpec((_T, 64), lambda p, ij: (ij[0, p], 0)),
            pl.BlockSpec((_T, 64), lambda p, ij: (ij[1, p], 0)),
            pl.BlockSpec((_T, 1), lambda p, ij: (ij[0, p], 0)),
            pl.BlockSpec((1, _T), lambda p, ij: (0, ij[1, p])),
        ],
        out_specs=pl.BlockSpec((1, 8), lambda p, ij: (0, 0)),
        scratch_shapes=[pltpu.VMEM((1, _N), jnp.float32)] * 3,
    )
    acc = pl.pallas_call(
        _tl_kernel,
        grid_spec=grid_spec,
        out_shape=jax.ShapeDtypeStruct((1, 8), jnp.float32),
    )(jnp.asarray(_IJ), x, x, t32.reshape(_N, 1), t32.reshape(1, _N))
    return acc[0]


def kernel(inputs, targets):
    t32 = targets.astype(jnp.int32)
    s = _triplet_stats(inputs, t32)
    n = jnp.float32(_N)
    return (s[0] / n, s[1] / n, s[2] / n, s[3] / n,
            s[4].astype(jnp.int32), s[5] / n, s[6] / n, s[7] / n)
